# baseline (speedup 1.0000x reference)
import jax
import jax.numpy as jnp
from jax import lax
from jax.experimental import pallas as pl
from jax.experimental.pallas import tpu as pltpu
from jax.experimental.pallas import tpu_sc as plsc

N_NODES = 10000
N_EDGES = 320000
HIDDEN = 128
EDGE_DIM = 16

NC = 2
NS = 16
NW = NC * NS
E_PAD = 327680
CHUNK = 80
NCHUNK = E_PAD // (NW * CHUNK)
AGG_ROWS = 10240
ROWS_PER_TILE = AGG_ROWS // NS


def _edge_mlp_body(a_ref, w_ref, b_ref, o_ref):
    o_ref[...] = (
        jnp.dot(a_ref[...], w_ref[...], preferred_element_type=jnp.float32)
        + b_ref[...]
    )


def _edge_mlp(edge_attr, W_e, b_e):
    be = 4096
    grid = E_PAD // be
    return pl.pallas_call(
        _edge_mlp_body,
        grid=(grid,),
        in_specs=[
            pl.BlockSpec((be, EDGE_DIM), lambda i: (i, 0)),
            pl.BlockSpec((EDGE_DIM, HIDDEN), lambda i: (0, 0)),
            pl.BlockSpec((1, HIDDEN), lambda i: (0, 0)),
        ],
        out_specs=pl.BlockSpec((be, HIDDEN), lambda i: (i, 0)),
        out_shape=jax.ShapeDtypeStruct((E_PAD, HIDDEN), jnp.float32),
    )(edge_attr, W_e, b_e.reshape(1, HIDDEN))


def _sc_body(x_hbm, src_hbm, dst_hbm, e_hbm, out_hbm,
             srcb0, srcb1, dstb0, dstb1, ebuf0, ebuf1, xbuf0, xbuf1,
             agg, esem0, esem1, isem0, isem1):
    cid = lax.axis_index("c")
    sid = lax.axis_index("s")
    wid = cid * NS + sid
    base = wid * (NCHUNK * CHUNK)

    zero16 = jnp.zeros((16,), jnp.float32)

    def zfill(i, _):
        for k in range(HIDDEN // 16):
            ebuf0[i, pl.ds(k * 16, 16)] = zero16
        return 0

    lax.fori_loop(0, CHUNK, zfill, 0)
    for t in range(ROWS_PER_TILE // CHUNK):
        pltpu.sync_copy(
            ebuf0, agg.at[pl.ds(sid * ROWS_PER_TILE + t * CHUNK, CHUNK)]
        )
    plsc.subcore_barrier()

    bufs = (
        (ebuf0, xbuf0, srcb0, dstb0, esem0, isem0),
        (ebuf1, xbuf1, srcb1, dstb1, esem1, isem1),
    )

    def issue_idx(c, b):
        _, _, sb, db, _, isem = bufs[b]
        pltpu.async_copy(src_hbm.at[wid].at[c], sb, isem)
        pltpu.async_copy(dst_hbm.at[wid].at[c], db, isem)

    def wait_idx(b):
        _, _, sb, db, _, isem = bufs[b]
        pltpu.make_async_copy(src_hbm.at[0].at[0], sb, isem).wait()
        pltpu.make_async_copy(src_hbm.at[0].at[0], db, isem).wait()

    def issue_data(c, b):
        eb, xb, sb, _, esem, _ = bufs[b]
        pltpu.async_copy(e_hbm.at[pl.ds(base + c * CHUNK, CHUNK)], eb, esem)
        pltpu.async_copy(x_hbm.at[sb], xb, esem)

    def process(c, b):
        eb, xb, _, db, esem, _ = bufs[b]
        pltpu.make_async_copy(e_hbm.at[pl.ds(0, CHUNK)], eb, esem).wait()
        pltpu.make_async_copy(e_hbm.at[pl.ds(0, CHUNK)], xb, esem).wait()

        def erow(i, _):
            for k in range(HIDDEN // 16):
                sl = pl.ds(k * 16, 16)
                eb[i, sl] = jnp.maximum(eb[i, sl] + xb[i, sl], 0.0)
            return 0

        lax.fori_loop(0, CHUNK, erow, 0)
        pltpu.sync_copy(eb, agg.at[db], add=True)

    issue_idx(0, 0)
    issue_idx(1, 1)
    wait_idx(0)
    issue_data(0, 0)

    def half(c, b):
        @pl.when(c + 1 < NCHUNK)
        def _():
            wait_idx(1 - b)
            issue_data(c + 1, 1 - b)

        process(c, b)

        @pl.when(c + 2 < NCHUNK)
        def _():
            issue_idx(c + 2, b)

    def pair_body(i, _):
        half(i * 2, 0)
        half(i * 2 + 1, 1)
        return 0

    lax.fori_loop(0, NCHUNK // 2, pair_body, 0)
    plsc.subcore_barrier()

    rb = pl.ds(sid * ROWS_PER_TILE, ROWS_PER_TILE)
    pltpu.sync_copy(agg.at[rb], out_hbm.at[cid].at[rb])


def _sc_aggregate(x, src3, dst3, e):
    mesh = plsc.VectorSubcoreMesh(core_axis_name="c", subcore_axis_name="s")
    k = pl.kernel(
        _sc_body,
        out_type=jax.ShapeDtypeStruct((NC, AGG_ROWS, HIDDEN), jnp.float32),
        mesh=mesh,
        scratch_types=[
            pltpu.VMEM((CHUNK,), jnp.int32),
            pltpu.VMEM((CHUNK,), jnp.int32),
            pltpu.VMEM((CHUNK,), jnp.int32),
            pltpu.VMEM((CHUNK,), jnp.int32),
            pltpu.VMEM((CHUNK, HIDDEN), jnp.float32),
            pltpu.VMEM((CHUNK, HIDDEN), jnp.float32),
            pltpu.VMEM((CHUNK, HIDDEN), jnp.float32),
            pltpu.VMEM((CHUNK, HIDDEN), jnp.float32),
            pltpu.VMEM_SHARED((AGG_ROWS, HIDDEN), jnp.float32),
            pltpu.SemaphoreType.DMA,
            pltpu.SemaphoreType.DMA,
            pltpu.SemaphoreType.DMA,
            pltpu.SemaphoreType.DMA,
        ],
    )
    return k(x, src3, dst3, e)


def _node_mlp_body(x_ref, a_ref, w1_ref, b1_ref, w2_ref, b2_ref,
                   g_ref, bt_ref, o_ref):
    x = x_ref[...]
    h = x + a_ref[0] + a_ref[1]
    h = jnp.maximum(
        jnp.dot(h, w1_ref[...], preferred_element_type=jnp.float32)
        + b1_ref[...], 0.0)
    h = (jnp.dot(h, w2_ref[...], preferred_element_type=jnp.float32)
         + b2_ref[...])
    mean = jnp.mean(h, axis=0, keepdims=True)
    var = jnp.mean((h - mean) ** 2, axis=0, keepdims=True)
    h = (h - mean) * lax.rsqrt(var + 1e-5) * g_ref[...] + bt_ref[...]
    o_ref[...] = jnp.maximum(h, 0.0) + x


def _node_mlp(x, aggs, W1, b1, W2, b2, gamma, beta):
    return pl.pallas_call(
        _node_mlp_body,
        grid=(1,),
        in_specs=[
            pl.BlockSpec((N_NODES, HIDDEN), lambda i: (0, 0)),
            pl.BlockSpec((NC, N_NODES, HIDDEN), lambda i: (0, 0, 0)),
            pl.BlockSpec((HIDDEN, HIDDEN), lambda i: (0, 0)),
            pl.BlockSpec((1, HIDDEN), lambda i: (0, 0)),
            pl.BlockSpec((HIDDEN, HIDDEN), lambda i: (0, 0)),
            pl.BlockSpec((1, HIDDEN), lambda i: (0, 0)),
            pl.BlockSpec((1, HIDDEN), lambda i: (0, 0)),
            pl.BlockSpec((1, HIDDEN), lambda i: (0, 0)),
        ],
        out_specs=pl.BlockSpec((N_NODES, HIDDEN), lambda i: (0, 0)),
        out_shape=jax.ShapeDtypeStruct((N_NODES, HIDDEN), jnp.float32),
    )(x, aggs, W1, b1.reshape(1, HIDDEN), W2, b2.reshape(1, HIDDEN),
      gamma.reshape(1, HIDDEN), beta.reshape(1, HIDDEN))


def kernel(x, edge_index, edge_attr, W_e, b_e, W1, b1, W2, b2, gamma, beta):
    npad = E_PAD - N_EDGES
    src = jnp.concatenate(
        [edge_index[0].astype(jnp.int32), jnp.zeros((npad,), jnp.int32)])
    dst = jnp.concatenate(
        [edge_index[1].astype(jnp.int32),
         jnp.full((npad,), N_NODES, jnp.int32)])
    src3 = src.reshape(NW, NCHUNK, CHUNK)
    dst3 = dst.reshape(NW, NCHUNK, CHUNK)
    ea_pad = jnp.concatenate(
        [edge_attr, jnp.zeros((npad, EDGE_DIM), jnp.float32)])
    e = _edge_mlp(ea_pad, W_e, b_e)
    aggs = _sc_aggregate(x, src3, dst3, e)
    return _node_mlp(x, aggs, W1, b1, W2, b2, gamma, beta)


# speedup vs baseline: 1.6937x; 1.6937x over previous
import jax
import jax.numpy as jnp
from jax import lax
from jax.experimental import pallas as pl
from jax.experimental.pallas import tpu as pltpu
from jax.experimental.pallas import tpu_sc as plsc

N_NODES = 10000
N_EDGES = 320000
HIDDEN = 128
EDGE_DIM = 16

NC = 2
NS = 16
NW = NC * NS
CHUNK = 80
PER_TILE = N_EDGES // NW
NCHUNK = PER_TILE // CHUNK
AGG_ROWS = 10240
ROWS_PER_TILE = AGG_ROWS // NS


def _edge_mlp_body(a_ref, w_ref, b_ref, o_ref):
    o_ref[...] = (
        jnp.dot(a_ref[...], w_ref[...], preferred_element_type=jnp.float32)
        + b_ref[...]
    )


def _edge_mlp(edge_attr, W_e, b_e):
    be = 4000
    grid = N_EDGES // be
    return pl.pallas_call(
        _edge_mlp_body,
        grid=(grid,),
        in_specs=[
            pl.BlockSpec((be, EDGE_DIM), lambda i: (i, 0)),
            pl.BlockSpec((EDGE_DIM, HIDDEN), lambda i: (0, 0)),
            pl.BlockSpec((1, HIDDEN), lambda i: (0, 0)),
        ],
        out_specs=pl.BlockSpec((be, HIDDEN), lambda i: (i, 0)),
        out_shape=jax.ShapeDtypeStruct((N_EDGES, HIDDEN), jnp.float32),
    )(edge_attr, W_e, b_e.reshape(1, HIDDEN))


def _sc_body(x_hbm, src_hbm, dst_hbm, e_hbm, out_hbm,
             srcb0, srcb1, dstb0, dstb1, ebuf0, ebuf1, xbuf0, xbuf1,
             agg, esem0, esem1, isem0, isem1):
    cid = lax.axis_index("c")
    sid = lax.axis_index("s")
    wid = cid * NS + sid
    base = wid * PER_TILE

    zero16 = jnp.zeros((16,), jnp.float32)

    def zfill(i, _):
        for k in range(HIDDEN // 16):
            ebuf0[i, pl.ds(k * 16, 16)] = zero16
        return 0

    lax.fori_loop(0, CHUNK, zfill, 0)
    for t in range(ROWS_PER_TILE // CHUNK):
        pltpu.sync_copy(
            ebuf0, agg.at[pl.ds(sid * ROWS_PER_TILE + t * CHUNK, CHUNK)]
        )
    plsc.subcore_barrier()

    bufs = (
        (ebuf0, xbuf0, srcb0, dstb0, esem0, isem0),
        (ebuf1, xbuf1, srcb1, dstb1, esem1, isem1),
    )

    def issue_idx(c, b):
        _, _, sb, db, _, isem = bufs[b]
        pltpu.async_copy(src_hbm.at[wid].at[c], sb, isem)
        pltpu.async_copy(dst_hbm.at[wid].at[c], db, isem)

    def wait_idx(b):
        _, _, sb, db, _, isem = bufs[b]
        pltpu.make_async_copy(src_hbm.at[0].at[0], sb, isem).wait()
        pltpu.make_async_copy(src_hbm.at[0].at[0], db, isem).wait()

    def issue_data(c, b):
        eb, xb, sb, _, esem, _ = bufs[b]
        pltpu.async_copy(e_hbm.at[pl.ds(base + c * CHUNK, CHUNK)], eb, esem)
        pltpu.async_copy(x_hbm.at[sb], xb, esem)

    def process(c, b):
        eb, xb, _, db, esem, _ = bufs[b]
        pltpu.make_async_copy(e_hbm.at[pl.ds(0, CHUNK)], eb, esem).wait()
        pltpu.make_async_copy(e_hbm.at[pl.ds(0, CHUNK)], xb, esem).wait()

        def erow(i, _):
            for k in range(HIDDEN // 16):
                sl = pl.ds(k * 16, 16)
                eb[i, sl] = jnp.maximum(eb[i, sl] + xb[i, sl], 0.0)
            return 0

        lax.fori_loop(0, CHUNK, erow, 0)
        pltpu.sync_copy(eb, agg.at[db], add=True)

    issue_idx(0, 0)
    issue_idx(1, 1)
    wait_idx(0)
    issue_data(0, 0)

    def half(c, b, last=False):
        if not last:
            wait_idx(1 - b)
            issue_data(c + 1, 1 - b)

        process(c, b)

        @pl.when(c + 2 < NCHUNK)
        def _():
            issue_idx(c + 2, b)

    def pair_body(i, _):
        half(i * 2, 0)
        half(i * 2 + 1, 1)
        return 0

    lax.fori_loop(0, NCHUNK // 2, pair_body, 0)
    half(NCHUNK - 1, 0, last=True)
    plsc.subcore_barrier()

    rb = pl.ds(sid * ROWS_PER_TILE, ROWS_PER_TILE)
    pltpu.sync_copy(agg.at[rb], out_hbm.at[cid].at[rb])


def _sc_aggregate(x, src3, dst3, e):
    mesh = plsc.VectorSubcoreMesh(core_axis_name="c", subcore_axis_name="s")
    k = pl.kernel(
        _sc_body,
        out_type=jax.ShapeDtypeStruct((NC, AGG_ROWS, HIDDEN), jnp.float32),
        mesh=mesh,
        scratch_types=[
            pltpu.VMEM((CHUNK,), jnp.int32),
            pltpu.VMEM((CHUNK,), jnp.int32),
            pltpu.VMEM((CHUNK,), jnp.int32),
            pltpu.VMEM((CHUNK,), jnp.int32),
            pltpu.VMEM((CHUNK, HIDDEN), jnp.float32),
            pltpu.VMEM((CHUNK, HIDDEN), jnp.float32),
            pltpu.VMEM((CHUNK, HIDDEN), jnp.float32),
            pltpu.VMEM((CHUNK, HIDDEN), jnp.float32),
            pltpu.VMEM_SHARED((AGG_ROWS, HIDDEN), jnp.float32),
            pltpu.SemaphoreType.DMA,
            pltpu.SemaphoreType.DMA,
            pltpu.SemaphoreType.DMA,
            pltpu.SemaphoreType.DMA,
        ],
    )
    return k(x, src3, dst3, e)


def _node_mlp_body(x_ref, a_ref, w1_ref, b1_ref, w2_ref, b2_ref,
                   g_ref, bt_ref, o_ref):
    x = x_ref[...]
    h = x + a_ref[0] + a_ref[1]
    h = jnp.maximum(
        jnp.dot(h, w1_ref[...], preferred_element_type=jnp.float32)
        + b1_ref[...], 0.0)
    h = (jnp.dot(h, w2_ref[...], preferred_element_type=jnp.float32)
         + b2_ref[...])
    mean = jnp.mean(h, axis=0, keepdims=True)
    var = jnp.mean((h - mean) ** 2, axis=0, keepdims=True)
    h = (h - mean) * lax.rsqrt(var + 1e-5) * g_ref[...] + bt_ref[...]
    o_ref[...] = jnp.maximum(h, 0.0) + x


def _node_mlp(x, aggs, W1, b1, W2, b2, gamma, beta):
    return pl.pallas_call(
        _node_mlp_body,
        grid=(1,),
        in_specs=[
            pl.BlockSpec((N_NODES, HIDDEN), lambda i: (0, 0)),
            pl.BlockSpec((NC, N_NODES, HIDDEN), lambda i: (0, 0, 0)),
            pl.BlockSpec((HIDDEN, HIDDEN), lambda i: (0, 0)),
            pl.BlockSpec((1, HIDDEN), lambda i: (0, 0)),
            pl.BlockSpec((HIDDEN, HIDDEN), lambda i: (0, 0)),
            pl.BlockSpec((1, HIDDEN), lambda i: (0, 0)),
            pl.BlockSpec((1, HIDDEN), lambda i: (0, 0)),
            pl.BlockSpec((1, HIDDEN), lambda i: (0, 0)),
        ],
        out_specs=pl.BlockSpec((N_NODES, HIDDEN), lambda i: (0, 0)),
        out_shape=jax.ShapeDtypeStruct((N_NODES, HIDDEN), jnp.float32),
    )(x, aggs, W1, b1.reshape(1, HIDDEN), W2, b2.reshape(1, HIDDEN),
      gamma.reshape(1, HIDDEN), beta.reshape(1, HIDDEN))


def kernel(x, edge_index, edge_attr, W_e, b_e, W1, b1, W2, b2, gamma, beta):
    src3 = edge_index[0].astype(jnp.int32).reshape(NW, NCHUNK, CHUNK)
    dst3 = edge_index[1].astype(jnp.int32).reshape(NW, NCHUNK, CHUNK)
    e = _edge_mlp(edge_attr, W_e, b_e)
    aggs = _sc_aggregate(x, src3, dst3, e)
    return _node_mlp(x, aggs, W1, b1, W2, b2, gamma, beta)


# speedup vs baseline: 1.7756x; 1.0484x over previous
import jax
import jax.numpy as jnp
from jax import lax
from jax.experimental import pallas as pl
from jax.experimental.pallas import tpu as pltpu
from jax.experimental.pallas import tpu_sc as plsc

N_NODES = 10000
N_EDGES = 320000
HIDDEN = 128
EDGE_DIM = 16

NC = 2
NS = 16
NW = NC * NS
CHUNK = 80
PER_TILE = N_EDGES // NW
NCHUNK = PER_TILE // CHUNK
AGG_ROWS = 10240
ROWS_PER_TILE = AGG_ROWS // NS


def _edge_mlp_body(a_ref, w_ref, b_ref, o_ref):
    o_ref[...] = (
        jnp.dot(a_ref[...], w_ref[...], preferred_element_type=jnp.float32)
        + b_ref[...]
    )


def _edge_mlp(edge_attr, W_e, b_e):
    be = 8000
    grid = N_EDGES // be
    return pl.pallas_call(
        _edge_mlp_body,
        grid=(grid,),
        in_specs=[
            pl.BlockSpec((be, EDGE_DIM), lambda i: (i, 0)),
            pl.BlockSpec((EDGE_DIM, HIDDEN), lambda i: (0, 0)),
            pl.BlockSpec((1, HIDDEN), lambda i: (0, 0)),
        ],
        out_specs=pl.BlockSpec((be, HIDDEN), lambda i: (i, 0)),
        out_shape=jax.ShapeDtypeStruct((N_EDGES, HIDDEN), jnp.float32),
    )(edge_attr, W_e, b_e.reshape(1, HIDDEN))


def _sc_body(x_hbm, src_hbm, dst_hbm, e_hbm, out_hbm,
             srcb0, srcb1, dstb0, dstb1, ebuf0, ebuf1, xbuf0, xbuf1,
             agg, esem0, esem1, isem0, isem1):
    cid = lax.axis_index("c")
    sid = lax.axis_index("s")
    wid = cid * NS + sid
    base = wid * PER_TILE

    zero16 = jnp.zeros((16,), jnp.float32)

    def zfill(i, _):
        for k in range(HIDDEN // 16):
            ebuf0[i, pl.ds(k * 16, 16)] = zero16
        return 0

    lax.fori_loop(0, CHUNK, zfill, 0)
    for t in range(ROWS_PER_TILE // CHUNK):
        pltpu.sync_copy(
            ebuf0, agg.at[pl.ds(sid * ROWS_PER_TILE + t * CHUNK, CHUNK)]
        )
    plsc.subcore_barrier()

    bufs = (
        (ebuf0, xbuf0, srcb0, dstb0, esem0, isem0),
        (ebuf1, xbuf1, srcb1, dstb1, esem1, isem1),
    )

    def issue_idx(c, b):
        _, _, sb, db, _, isem = bufs[b]
        off = base + c * CHUNK
        pltpu.async_copy(src_hbm.at[pl.ds(off, CHUNK)], sb, isem)
        pltpu.async_copy(dst_hbm.at[pl.ds(off, CHUNK)], db, isem)

    def wait_idx(b):
        _, _, sb, db, _, isem = bufs[b]
        pltpu.make_async_copy(src_hbm.at[pl.ds(0, CHUNK)], sb, isem).wait()
        pltpu.make_async_copy(src_hbm.at[pl.ds(0, CHUNK)], db, isem).wait()

    def issue_data(c, b):
        eb, xb, sb, _, esem, _ = bufs[b]
        pltpu.async_copy(e_hbm.at[pl.ds(base + c * CHUNK, CHUNK)], eb, esem)
        pltpu.async_copy(x_hbm.at[sb], xb, esem)

    def process(c, b):
        eb, xb, _, db, esem, _ = bufs[b]
        pltpu.make_async_copy(e_hbm.at[pl.ds(0, CHUNK)], eb, esem).wait()
        pltpu.make_async_copy(e_hbm.at[pl.ds(0, CHUNK)], xb, esem).wait()

        def erow(i, _):
            for k in range(HIDDEN // 16):
                sl = pl.ds(k * 16, 16)
                eb[i, sl] = jnp.maximum(eb[i, sl] + xb[i, sl], 0.0)
            return 0

        lax.fori_loop(0, CHUNK, erow, 0)
        pltpu.sync_copy(eb, agg.at[db], add=True)

    issue_idx(0, 0)
    issue_idx(1, 1)
    wait_idx(0)
    issue_data(0, 0)

    def half(c, b, last=False):
        if not last:
            wait_idx(1 - b)
            issue_data(c + 1, 1 - b)

        process(c, b)

        @pl.when(c + 2 < NCHUNK)
        def _():
            issue_idx(c + 2, b)

    def pair_body(i, _):
        half(i * 2, 0)
        half(i * 2 + 1, 1)
        return 0

    lax.fori_loop(0, NCHUNK // 2, pair_body, 0)
    half(NCHUNK - 1, 0, last=True)
    plsc.subcore_barrier()

    rb = pl.ds(sid * ROWS_PER_TILE, ROWS_PER_TILE)
    pltpu.sync_copy(agg.at[rb], out_hbm.at[cid].at[rb])


def _sc_aggregate(x, src3, dst3, e):
    mesh = plsc.VectorSubcoreMesh(core_axis_name="c", subcore_axis_name="s")
    k = pl.kernel(
        _sc_body,
        out_type=jax.ShapeDtypeStruct((NC, AGG_ROWS, HIDDEN), jnp.float32),
        mesh=mesh,
        scratch_types=[
            pltpu.VMEM((CHUNK,), jnp.int32),
            pltpu.VMEM((CHUNK,), jnp.int32),
            pltpu.VMEM((CHUNK,), jnp.int32),
            pltpu.VMEM((CHUNK,), jnp.int32),
            pltpu.VMEM((CHUNK, HIDDEN), jnp.float32),
            pltpu.VMEM((CHUNK, HIDDEN), jnp.float32),
            pltpu.VMEM((CHUNK, HIDDEN), jnp.float32),
            pltpu.VMEM((CHUNK, HIDDEN), jnp.float32),
            pltpu.VMEM_SHARED((AGG_ROWS, HIDDEN), jnp.float32),
            pltpu.SemaphoreType.DMA,
            pltpu.SemaphoreType.DMA,
            pltpu.SemaphoreType.DMA,
            pltpu.SemaphoreType.DMA,
        ],
    )
    return k(x, src3, dst3, e)


def _node_mlp_body(x_ref, a_ref, w1_ref, b1_ref, w2_ref, b2_ref,
                   g_ref, bt_ref, o_ref):
    x = x_ref[...]
    h = x + a_ref[0] + a_ref[1]
    h = jnp.maximum(
        jnp.dot(h, w1_ref[...], preferred_element_type=jnp.float32)
        + b1_ref[...], 0.0)
    h = (jnp.dot(h, w2_ref[...], preferred_element_type=jnp.float32)
         + b2_ref[...])
    mean = jnp.mean(h, axis=0, keepdims=True)
    var = jnp.mean((h - mean) ** 2, axis=0, keepdims=True)
    h = (h - mean) * lax.rsqrt(var + 1e-5) * g_ref[...] + bt_ref[...]
    o_ref[...] = jnp.maximum(h, 0.0) + x


def _node_mlp(x, aggs, W1, b1, W2, b2, gamma, beta):
    return pl.pallas_call(
        _node_mlp_body,
        grid=(1,),
        in_specs=[
            pl.BlockSpec((N_NODES, HIDDEN), lambda i: (0, 0)),
            pl.BlockSpec((NC, N_NODES, HIDDEN), lambda i: (0, 0, 0)),
            pl.BlockSpec((HIDDEN, HIDDEN), lambda i: (0, 0)),
            pl.BlockSpec((1, HIDDEN), lambda i: (0, 0)),
            pl.BlockSpec((HIDDEN, HIDDEN), lambda i: (0, 0)),
            pl.BlockSpec((1, HIDDEN), lambda i: (0, 0)),
            pl.BlockSpec((1, HIDDEN), lambda i: (0, 0)),
            pl.BlockSpec((1, HIDDEN), lambda i: (0, 0)),
        ],
        out_specs=pl.BlockSpec((N_NODES, HIDDEN), lambda i: (0, 0)),
        out_shape=jax.ShapeDtypeStruct((N_NODES, HIDDEN), jnp.float32),
    )(x, aggs, W1, b1.reshape(1, HIDDEN), W2, b2.reshape(1, HIDDEN),
      gamma.reshape(1, HIDDEN), beta.reshape(1, HIDDEN))


def kernel(x, edge_index, edge_attr, W_e, b_e, W1, b1, W2, b2, gamma, beta):
    src = edge_index[0].astype(jnp.int32)
    dst = edge_index[1].astype(jnp.int32)
    e = _edge_mlp(edge_attr, W_e, b_e)
    aggs = _sc_aggregate(x, src, dst, e)
    return _node_mlp(x, aggs, W1, b1, W2, b2, gamma, beta)
